# baseline (speedup 1.0000x reference)
import functools

import jax
import jax.numpy as jnp
from jax import lax
from jax.experimental import pallas as pl

D = 300
DP = 320
H = 160
ND = 10000
NDP = 10240
B = 1024
L = 200
C = 20
SCALE = float(D) ** -0.5


def _combine3(h1, h2, h3):
    hs = (h1, h2, h3)
    d = [[jnp.sum(hs[a] * hs[b], axis=1, keepdims=True) * SCALE for b in range(3)]
         for a in range(3)]
    w = [jnp.zeros_like(d[0][0]) for _ in range(3)]
    for a in range(3):
        m = jnp.maximum(jnp.maximum(d[a][0], d[a][1]), d[a][2])
        e = [jnp.exp(d[a][b] - m) for b in range(3)]
        tot = e[0] + e[1] + e[2]
        for b_ in range(3):
            w[b_] = w[b_] + e[b_] / tot
    return (w[0] * h1 + w[1] * h2 + w[2] * h3) * (1.0 / 3.0)


def _word_combine_body(dstf_ref, sums_ref, w1_ref, w2_ref, gwe0_ref, gwe1_ref):
    i = pl.program_id(0)
    rows = dstf_ref.shape[1]
    dstf = jnp.concatenate([dstf_ref[0], dstf_ref[1]], axis=-1)
    hs = []
    for r in range(3):
        s = jnp.concatenate([sums_ref[r, 0], sums_ref[r, 1]], axis=-1)
        cnt = jnp.maximum(s[:, DP - 1:DP], 1.0)
        mean = s / cnt
        h = (jnp.dot(dstf, w1_ref[r], preferred_element_type=jnp.float32)
             + jnp.dot(mean, w2_ref[r], preferred_element_type=jnp.float32))
        hs.append(h)
    doc = _combine3(*hs)
    row_id = i * rows + lax.broadcasted_iota(jnp.int32, (rows, DP), 0)
    col_id = lax.broadcasted_iota(jnp.int32, (rows, DP), 1)
    base = jnp.where(row_id < ND, doc + dstf, 0.0)
    gwe = jnp.where((col_id == DP - 1) & (row_id <= ND), 1.0, base)
    gwe0_ref[...] = gwe[:, :H]
    gwe1_ref[...] = gwe[:, H:]


def _word_combine(dstf, sums, w1, w2, rows=512):
    grid = (NDP // rows,)
    return pl.pallas_call(
        _word_combine_body,
        grid=grid,
        in_specs=[
            pl.BlockSpec((2, rows, H), lambda i: (0, i, 0)),
            pl.BlockSpec((3, 2, rows, H), lambda i: (0, 0, i, 0)),
            pl.BlockSpec((3, DP, DP), lambda i: (0, 0, 0)),
            pl.BlockSpec((3, DP, DP), lambda i: (0, 0, 0)),
        ],
        out_specs=[
            pl.BlockSpec((rows, H), lambda i: (i, 0)),
            pl.BlockSpec((rows, H), lambda i: (i, 0)),
        ],
        out_shape=[
            jax.ShapeDtypeStruct((NDP, H), jnp.float32),
            jax.ShapeDtypeStruct((NDP, H), jnp.float32),
        ],
    )(dstf, sums, w1, w2)


def _final_body(seqsum_ref, docsums_ref, wd_ref, w1d_ref, w2d_ref, wfc_ref, out_ref):
    rows = seqsum_ref.shape[1]
    seqsum = jnp.concatenate([seqsum_ref[0], seqsum_ref[1]], axis=-1)
    doc_out = jnp.dot(seqsum, wd_ref[...], preferred_element_type=jnp.float32) * (1.0 / L)
    col_id = lax.broadcasted_iota(jnp.int32, (rows, DP), 1)
    dv = jnp.where(col_id == DP - 1, 1.0, doc_out)
    hs = []
    for r in range(3):
        s = jnp.concatenate([docsums_ref[r, 0], docsums_ref[r, 1]], axis=-1)
        cnt = jnp.maximum(s[:, DP - 1:DP], 1.0)
        mean = s / cnt
        h = (jnp.dot(dv, w1d_ref[r], preferred_element_type=jnp.float32)
             + jnp.dot(mean, w2d_ref[r], preferred_element_type=jnp.float32))
        hs.append(h)
    gnn = _combine3(*hs)
    resid = gnn + dv
    out_ref[...] = jnp.dot(resid, wfc_ref[...], preferred_element_type=jnp.float32)


def _final(seqsum, docsums, wd, w1d, w2d, wfc, rows=512):
    grid = (B // rows,)
    return pl.pallas_call(
        _final_body,
        grid=grid,
        in_specs=[
            pl.BlockSpec((2, rows, H), lambda i: (0, i, 0)),
            pl.BlockSpec((3, 2, rows, H), lambda i: (0, 0, i, 0)),
            pl.BlockSpec((DP, DP), lambda i: (0, 0)),
            pl.BlockSpec((3, DP, DP), lambda i: (0, 0, 0)),
            pl.BlockSpec((3, DP, DP), lambda i: (0, 0, 0)),
            pl.BlockSpec((DP, 128), lambda i: (0, 0)),
        ],
        out_specs=pl.BlockSpec((rows, 128), lambda i: (i, 0)),
        out_shape=jax.ShapeDtypeStruct((B, 128), jnp.float32),
    )(seqsum, docsums, wd, w1d, w2d, wfc)


def _pad_w(W, b):
    w1 = jnp.zeros((DP, DP), jnp.float32).at[:D, :D].set(W[:D]).at[DP - 1, :D].set(b)
    w2 = jnp.zeros((DP, DP), jnp.float32).at[:D, :D].set(W[D:])
    return w1, w2


def _halves(emb):
    V = emb.shape[0]
    h0 = emb[:, :H]
    h1 = jnp.concatenate(
        [emb[:, H:D], jnp.zeros((V, H - (D - H) - 1), jnp.float32),
         jnp.ones((V, 1), jnp.float32)], axis=1)
    return h0, h1


def kernel(dst_nids, src_nids_dis, src_nids_pmi, src_nids_top, src_idx_dis, dst_idx_dis, src_idx_pmi, dst_idx_pmi, src_idx_top, dst_idx_top, src_nids_dis_doc, src_nids_pmi_doc, src_nids_top_doc, src_idx_dis_doc, dst_idx_dis_doc, src_idx_pmi_doc, dst_idx_pmi_doc, src_idx_top_doc, dst_idx_top_doc, x_batch, length_batch, return_doc_representation, emb_word, emb_doc, W_dis, b_dis, W_pmi, b_pmi, W_top, b_top, W_dis_d, b_dis_d, W_pmi_d, b_pmi_d, W_top_d, b_top_d, W_dense, b_dense, W_fc, b_fc):
    wt = _halves(emb_word)
    dt = _halves(emb_doc)

    def seg(table_h, gidx, sidx, nseg):
        return jax.ops.segment_sum(jnp.take(table_h, gidx, axis=0), sidx,
                                   num_segments=nseg)

    sums_w = []
    for src_nids, sidx, didx in (
            (src_nids_dis, src_idx_dis, dst_idx_dis),
            (src_nids_pmi, src_idx_pmi, dst_idx_pmi),
            (src_nids_top, src_idx_top, dst_idx_top)):
        nid = jnp.take(src_nids, sidx, axis=0)
        sums_w.append(jnp.stack([seg(wt[0], nid, didx, NDP),
                                 seg(wt[1], nid, didx, NDP)]))
    sums_w = jnp.stack(sums_w)

    dstf = jnp.stack([
        jnp.zeros((NDP, H), jnp.float32).at[:ND].set(jnp.take(wt[0], dst_nids, axis=0)),
        jnp.zeros((NDP, H), jnp.float32).at[:ND].set(jnp.take(wt[1], dst_nids, axis=0)),
    ])

    w1s, w2s = [], []
    for W, b_ in ((W_dis, b_dis), (W_pmi, b_pmi), (W_top, b_top)):
        a, b2 = _pad_w(W, b_)
        w1s.append(a)
        w2s.append(b2)
    gwe0, gwe1 = _word_combine(dstf, sums_w, jnp.stack(w1s), jnp.stack(w2s))

    x_flat = x_batch.reshape(-1)
    seg_x = jnp.repeat(jnp.arange(B, dtype=jnp.int32), L)
    seqsum = jnp.stack([seg(gwe0, x_flat, seg_x, B), seg(gwe1, x_flat, seg_x, B)])

    sums_d = []
    for src_nids, sidx, didx in (
            (src_nids_dis_doc, src_idx_dis_doc, dst_idx_dis_doc),
            (src_nids_pmi_doc, src_idx_pmi_doc, dst_idx_pmi_doc),
            (src_nids_top_doc, src_idx_top_doc, dst_idx_top_doc)):
        nid = jnp.take(src_nids, sidx, axis=0)
        sums_d.append(jnp.stack([seg(dt[0], nid, didx, B),
                                 seg(dt[1], nid, didx, B)]))
    sums_d = jnp.stack(sums_d)

    wd = jnp.zeros((DP, DP), jnp.float32).at[:D, :D].set(W_dense).at[DP - 1, :D].set(b_dense)
    w1d, w2d = [], []
    for W, b_ in ((W_dis_d, b_dis_d), (W_pmi_d, b_pmi_d), (W_top_d, b_top_d)):
        a, b2 = _pad_w(W, b_)
        w1d.append(a)
        w2d.append(b2)
    wfc = jnp.zeros((DP, 128), jnp.float32).at[:D, :C].set(W_fc).at[DP - 1, :C].set(b_fc)

    out = _final(seqsum, sums_d, wd, jnp.stack(w1d), jnp.stack(w2d), wfc)
    return out[:, :C]
